# baseline (speedup 1.0000x reference)
import functools

import jax
import jax.numpy as jnp
from jax import lax
from jax.experimental import pallas as pl
from jax.experimental.pallas import tpu as pltpu
from jax.experimental.pallas import tpu_sc as plsc

N = 10000
E = 320000
D = 128

NC = 2
NS = 16
NW = NC * NS
EPW = E // NW
CH = 128
NCHUNK = EPW // CH
CHT = EPW - NCHUNK * CH
NPAD = 10112
RPT = NPAD // NS

@functools.cache
def _spmv_sc_build():
    mesh = plsc.VectorSubcoreMesh(
        core_axis_name="c", subcore_axis_name="s",
        num_cores=NC, num_subcores=NS)
    return pl.kernel(
        _spmv_sc,
        out_type=jax.ShapeDtypeStruct((NC * NPAD, D), jnp.float32),
        mesh=mesh,
        scratch_types=[
        pltpu.VMEM((EPW,), jnp.int32),
        [pltpu.VMEM((CH,), jnp.int32) for _ in range(2)],
        [pltpu.VMEM((CH, D), jnp.float32) for _ in range(2)],
        pltpu.VMEM_SHARED((NPAD, D), jnp.float32),
        [pltpu.SemaphoreType.DMA for _ in range(2)],
        [pltpu.SemaphoreType.DMA for _ in range(2)],
        [pltpu.SemaphoreType.DMA for _ in range(2)],
        ],
    )


def _spmv_sc(h_hbm, src_hbm, dst_hbm, zero_hbm, out_hbm,
             src_v, dst_v, rows_v, acc_sh, gsem, dsem, ssem):
    cid = lax.axis_index("c")
    sid = lax.axis_index("s")
    wid = cid * NS + sid
    ebase = wid * EPW

    pltpu.sync_copy(src_hbm.at[pl.ds(ebase, EPW)], src_v)
    pltpu.sync_copy(zero_hbm.at[pl.ds(sid * RPT, RPT)],
                    acc_sh.at[pl.ds(sid * RPT, RPT)])
    plsc.subcore_barrier()

    def start(i, b):
        pltpu.async_copy(dst_hbm.at[pl.ds(ebase + i * CH, CH)],
                         dst_v[b], dsem[b])
        pltpu.async_copy(h_hbm.at[src_v.at[pl.ds(i * CH, CH)]],
                         rows_v[b], gsem[b])

    def wait_in(i, b):
        pltpu.make_async_copy(dst_hbm.at[pl.ds(ebase + i * CH, CH)],
                              dst_v[b], dsem[b]).wait()
        pltpu.make_async_copy(h_hbm.at[src_v.at[pl.ds(i * CH, CH)]],
                              rows_v[b], gsem[b]).wait()

    def scat_start(b):
        pltpu.async_copy(rows_v[b], acc_sh.at[dst_v[b]], ssem[b], add=True)

    def scat_wait(b):
        pltpu.make_async_copy(rows_v[b], acc_sh.at[dst_v[b]],
                              ssem[b]).wait()

    start(0, 0)
    start(1, 1)

    def body(k, _):
        i = 2 * k
        wait_in(i, 0)
        scat_start(0)
        wait_in(i + 1, 1)
        scat_start(1)
        scat_wait(0)
        start(i + 2, 0)
        scat_wait(1)
        start(i + 3, 1)
        return ()

    lax.fori_loop(0, NCHUNK // 2 - 1, body, ())
    i = NCHUNK - 2
    wait_in(i, 0)
    scat_start(0)
    wait_in(i + 1, 1)
    scat_start(1)
    scat_wait(0)
    scat_wait(1)

    tb = ebase + NCHUNK * CH
    pltpu.async_copy(dst_hbm.at[pl.ds(tb, CHT)], dst_v[0].at[pl.ds(0, CHT)],
                     dsem[0])
    pltpu.async_copy(h_hbm.at[src_v.at[pl.ds(NCHUNK * CH, CHT)]],
                     rows_v[0].at[pl.ds(0, CHT)], gsem[0])
    pltpu.make_async_copy(dst_hbm.at[pl.ds(tb, CHT)],
                          dst_v[0].at[pl.ds(0, CHT)], dsem[0]).wait()
    pltpu.make_async_copy(h_hbm.at[src_v.at[pl.ds(NCHUNK * CH, CHT)]],
                          rows_v[0].at[pl.ds(0, CHT)], gsem[0]).wait()
    pltpu.sync_copy(rows_v[0].at[pl.ds(0, CHT)],
                    acc_sh.at[dst_v[0].at[pl.ds(0, CHT)]], add=True)

    plsc.subcore_barrier()
    pltpu.sync_copy(acc_sh.at[pl.ds(sid * RPT, RPT)],
                    out_hbm.at[pl.ds(cid * NPAD + sid * RPT, RPT)])


def _spmv(h, src, dst, zeros):
    p = _spmv_sc_build()(h, src, dst, zeros)
    return p[:N], p[NPAD:NPAD + N]


_RBLK = 2000


def _make_dense_body(term_sizes, relu):
    def body(*refs):
        o_ref = refs[-1]
        b_ref = refs[-2]
        idx = 0
        acc = None
        for npart in term_sizes:
            xs = refs[idx:idx + npart]
            w_ref = refs[idx + npart]
            idx += npart + 1
            xsum = xs[0][...]
            for r in xs[1:]:
                xsum = xsum + r[...]
            t = jnp.dot(xsum, w_ref[...], preferred_element_type=jnp.float32)
            acc = t if acc is None else acc + t
        acc = acc + b_ref[...]
        if relu:
            acc = jnp.maximum(acc, 0.0)
        o_ref[...] = acc
    return body


def _dense(terms, b, relu):
    term_sizes = tuple(len(parts) for parts, _ in terms)
    in_specs = []
    args = []
    for parts, w in terms:
        for p in parts:
            in_specs.append(pl.BlockSpec((_RBLK, p.shape[1]), lambda i: (i, 0)))
            args.append(p)
        in_specs.append(
            pl.BlockSpec((w.shape[0], w.shape[1]), lambda i: (0, 0)))
        args.append(w)
    b2 = b[None, :]
    in_specs.append(pl.BlockSpec((1, b2.shape[1]), lambda i: (0, 0)))
    args.append(b2)
    h = terms[0][1].shape[1]
    return pl.pallas_call(
        _make_dense_body(term_sizes, relu),
        grid=(N // _RBLK,),
        in_specs=in_specs,
        out_specs=pl.BlockSpec((_RBLK, h), lambda i: (i, 0)),
        out_shape=jax.ShapeDtypeStruct((N, h), jnp.float32),
    )(*args)


def kernel(x, adj_a, adj_b, Wa0, ba0, Wa1, ba1, Wb0, bb0, Wb1, bb1,
           Wm, bm, Wo, bo):
    zeros = jnp.zeros((NPAD, D), jnp.float32)
    src_a, dst_a = adj_a[1], adj_a[0]
    src_b, dst_b = adj_b[1], adj_b[0]
    Wm0, Wm1, Wm2 = Wm[:D], Wm[D:2 * D], Wm[2 * D:]
    Wo_a, Wo_b = Wo[:D], Wo[D:]

    ha0 = _dense([([x], Wa0)], ba0, relu=True)
    pa0, pa1 = _spmv(ha0, src_a, dst_a, zeros)
    ha1 = _dense([([pa0, pa1], Wa1)], ba1, relu=True)
    qa0, qa1 = _spmv(ha1, src_a, dst_a, zeros)

    hb0 = _dense([([x], Wb0)], bb0, relu=True)
    pb0, pb1 = _spmv(hb0, src_b, dst_b, zeros)
    hb1 = _dense([([pb0, pb1], Wb1)], bb1, relu=True)
    qb0, qb1 = _spmv(hb1, src_b, dst_b, zeros)
    xb = _dense([([hb0], Wm0), ([hb1], Wm1), ([qb0, qb1], Wm2)], bm, relu=True)

    out = _dense([([qa0, qa1], Wo_a), ([xb], Wo_b)], bo, relu=False)
    return out


# speedup vs baseline: 1.3798x; 1.3798x over previous
import functools

import jax
import jax.numpy as jnp
from jax import lax
from jax.experimental import pallas as pl
from jax.experimental.pallas import tpu as pltpu
from jax.experimental.pallas import tpu_sc as plsc

N = 10000
E = 320000
D = 128

NC = 2
NS = 16
EPT = E // NS
CH = 128
NCHUNK = EPT // CH
TAIL = EPT - NCHUNK * CH
NPAD = 10112
RPT = NPAD // NS
UNROLL = 6


@functools.cache
def _dual_spmv_build():
    mesh = plsc.VectorSubcoreMesh(
        core_axis_name="c", subcore_axis_name="s",
        num_cores=NC, num_subcores=NS)
    return pl.kernel(
        _dual_spmv_sc,
        out_type=jax.ShapeDtypeStruct((NC * NPAD, D), jnp.float32),
        mesh=mesh,
        scratch_types=[
            [pltpu.VMEM((CH,), jnp.int32) for _ in range(3)],
            [pltpu.VMEM((CH,), jnp.int32) for _ in range(3)],
            [pltpu.VMEM((CH, D), jnp.float32) for _ in range(2)],
            pltpu.VMEM_SHARED((NPAD, D), jnp.float32),
            [pltpu.SemaphoreType.DMA for _ in range(3)],
            [pltpu.SemaphoreType.DMA for _ in range(3)],
            [pltpu.SemaphoreType.DMA for _ in range(2)],
        ],
    )


def _dual_spmv_sc(ha_hbm, srca_hbm, dsta_hbm, hb_hbm, srcb_hbm, dstb_hbm,
                  zero_hbm, out_hbm,
                  sidx, didx, rows, acc_sh, ssem, dsem, gsem):
    cid = lax.axis_index("c")
    sid = lax.axis_index("s")

    pltpu.sync_copy(zero_hbm.at[pl.ds(sid * RPT, RPT)],
                    acc_sh.at[pl.ds(sid * RPT, RPT)])
    plsc.subcore_barrier()

    def run_branch(h_hbm, src_hbm, dst_hbm):
        ebase = sid * EPT

        def fetch(i, ib):
            off = ebase + i * CH
            pltpu.async_copy(src_hbm.at[pl.ds(off, CH)], sidx[ib], ssem[ib])
            pltpu.async_copy(dst_hbm.at[pl.ds(off, CH)], didx[ib], dsem[ib])

        def fetch_wait(i, ib):
            off = ebase + i * CH
            pltpu.make_async_copy(src_hbm.at[pl.ds(off, CH)],
                                  sidx[ib], ssem[ib]).wait()
            pltpu.make_async_copy(dst_hbm.at[pl.ds(off, CH)],
                                  didx[ib], dsem[ib]).wait()

        def gstart(ib, rb):
            pltpu.async_copy(h_hbm.at[sidx[ib]], rows[rb], gsem[rb])

        def gwait(ib, rb):
            pltpu.make_async_copy(h_hbm.at[sidx[ib]],
                                  rows[rb], gsem[rb]).wait()

        def scat(ib, rb):
            pltpu.sync_copy(rows[rb], acc_sh.at[didx[ib]], add=True)

        fetch(0, 0)
        fetch(1, 1)
        fetch_wait(0, 0)
        gstart(0, 0)

        def body(k, _):
            for u in range(UNROLL):
                i = UNROLL * k + u
                fetch(i + 2, (u + 2) % 3)
                fetch_wait(i + 1, (u + 1) % 3)
                gstart((u + 1) % 3, (u + 1) % 2)
                gwait(u % 3, u % 2)
                scat(u % 3, u % 2)
            return ()

        nk = (NCHUNK - UNROLL) // UNROLL
        lax.fori_loop(0, nk, body, ())
        for i in range(NCHUNK - UNROLL, NCHUNK):
            u = i % UNROLL
            if i + 2 < NCHUNK:
                fetch(i + 2, (u + 2) % 3)
            if i + 1 < NCHUNK:
                fetch_wait(i + 1, (u + 1) % 3)
                gstart((u + 1) % 3, (u + 1) % 2)
            gwait(u % 3, u % 2)
            scat(u % 3, u % 2)

        toff = ebase + NCHUNK * CH
        pltpu.async_copy(src_hbm.at[pl.ds(toff, TAIL)],
                         sidx[0].at[pl.ds(0, TAIL)], ssem[0])
        pltpu.async_copy(dst_hbm.at[pl.ds(toff, TAIL)],
                         didx[0].at[pl.ds(0, TAIL)], dsem[0])
        pltpu.make_async_copy(src_hbm.at[pl.ds(toff, TAIL)],
                              sidx[0].at[pl.ds(0, TAIL)], ssem[0]).wait()
        pltpu.make_async_copy(dst_hbm.at[pl.ds(toff, TAIL)],
                              didx[0].at[pl.ds(0, TAIL)], dsem[0]).wait()
        pltpu.async_copy(h_hbm.at[sidx[0].at[pl.ds(0, TAIL)]],
                         rows[0].at[pl.ds(0, TAIL)], gsem[0])
        pltpu.make_async_copy(h_hbm.at[sidx[0].at[pl.ds(0, TAIL)]],
                              rows[0].at[pl.ds(0, TAIL)], gsem[0]).wait()
        pltpu.sync_copy(rows[0].at[pl.ds(0, TAIL)],
                        acc_sh.at[didx[0].at[pl.ds(0, TAIL)]], add=True)

    @pl.when(cid == 0)
    def _():
        run_branch(ha_hbm, srca_hbm, dsta_hbm)

    @pl.when(cid == 1)
    def _():
        run_branch(hb_hbm, srcb_hbm, dstb_hbm)

    plsc.subcore_barrier()
    pltpu.sync_copy(acc_sh.at[pl.ds(sid * RPT, RPT)],
                    out_hbm.at[pl.ds(cid * NPAD + sid * RPT, RPT)])


def _dual_spmv(ha, srca, dsta, hb, srcb, dstb, zeros):
    p = _dual_spmv_build()(ha, srca, dsta, hb, srcb, dstb, zeros)
    return p[:N], p[NPAD:NPAD + N]


_RBLK = 2000


def _pair_body(xa_ref, wa_ref, ba_ref, xb_ref, wb_ref, bb_ref,
               oa_ref, ob_ref):
    oa_ref[...] = jnp.maximum(
        jnp.dot(xa_ref[...], wa_ref[...],
                preferred_element_type=jnp.float32) + ba_ref[...], 0.0)
    ob_ref[...] = jnp.maximum(
        jnp.dot(xb_ref[...], wb_ref[...],
                preferred_element_type=jnp.float32) + bb_ref[...], 0.0)


def _dense_pair(xa, Wa, ba, xb, Wb, bb):
    xspec = pl.BlockSpec((_RBLK, D), lambda i: (i, 0))
    wspec = pl.BlockSpec((D, D), lambda i: (0, 0))
    bspec = pl.BlockSpec((1, D), lambda i: (0, 0))
    ospec = pl.BlockSpec((_RBLK, D), lambda i: (i, 0))
    oshape = jax.ShapeDtypeStruct((N, D), jnp.float32)
    return pl.pallas_call(
        _pair_body,
        grid=(N // _RBLK,),
        in_specs=[xspec, wspec, bspec, xspec, wspec, bspec],
        out_specs=[ospec, ospec],
        out_shape=[oshape, oshape],
    )(xa, Wa, ba[None, :], xb, Wb, bb[None, :])


def _final_body(xa_ref, h0_ref, h1_ref, qb_ref, wm0_ref, wm1_ref, wm2_ref,
                bm_ref, woa_ref, wob_ref, bo_ref, o_ref):
    xb = jnp.maximum(
        jnp.dot(h0_ref[...], wm0_ref[...], preferred_element_type=jnp.float32)
        + jnp.dot(h1_ref[...], wm1_ref[...],
                  preferred_element_type=jnp.float32)
        + jnp.dot(qb_ref[...], wm2_ref[...],
                  preferred_element_type=jnp.float32)
        + bm_ref[...], 0.0)
    o_ref[...] = (
        jnp.dot(xa_ref[...], woa_ref[...], preferred_element_type=jnp.float32)
        + jnp.dot(xb, wob_ref[...], preferred_element_type=jnp.float32)
        + bo_ref[...])


def _final(xa, h0, h1, qb, Wm0, Wm1, Wm2, bm, Woa, Wob, bo):
    xspec = pl.BlockSpec((_RBLK, D), lambda i: (i, 0))
    wspec = pl.BlockSpec((D, D), lambda i: (0, 0))
    bspec = pl.BlockSpec((1, D), lambda i: (0, 0))
    return pl.pallas_call(
        _final_body,
        grid=(N // _RBLK,),
        in_specs=[xspec, xspec, xspec, xspec, wspec, wspec, wspec, bspec,
                  wspec, wspec, bspec],
        out_specs=pl.BlockSpec((_RBLK, D), lambda i: (i, 0)),
        out_shape=jax.ShapeDtypeStruct((N, D), jnp.float32),
    )(xa, h0, h1, qb, Wm0, Wm1, Wm2, bm[None, :], Woa, Wob, bo[None, :])


def kernel(x, adj_a, adj_b, Wa0, ba0, Wa1, ba1, Wb0, bb0, Wb1, bb1,
           Wm, bm, Wo, bo):
    zeros = jnp.zeros((NPAD, D), jnp.float32)
    src_a, dst_a = adj_a[1], adj_a[0]
    src_b, dst_b = adj_b[1], adj_b[0]
    Wm0, Wm1, Wm2 = Wm[:D], Wm[D:2 * D], Wm[2 * D:]
    Wo_a, Wo_b = Wo[:D], Wo[D:]

    ha0, hb0 = _dense_pair(x, Wa0, ba0, x, Wb0, bb0)
    pa, pb = _dual_spmv(ha0, src_a, dst_a, hb0, src_b, dst_b, zeros)
    ha1, hb1 = _dense_pair(pa, Wa1, ba1, pb, Wb1, bb1)
    xa, qb = _dual_spmv(ha1, src_a, dst_a, hb1, src_b, dst_b, zeros)
    return _final(xa, hb0, hb1, qb, Wm0, Wm1, Wm2, bm, Wo_a, Wo_b, bo)


# speedup vs baseline: 1.4678x; 1.0637x over previous
import functools

import jax
import jax.numpy as jnp
from jax import lax
from jax.experimental import pallas as pl
from jax.experimental.pallas import tpu as pltpu
from jax.experimental.pallas import tpu_sc as plsc

N = 10000
E = 320000
D = 128

NC = 2
NS = 16
EPT = E // NS
CH = 128
NCHUNK = EPT // CH
TAIL = EPT - NCHUNK * CH
NPAD = 10112
RPT = NPAD // NS
RLAST = N - (NS - 1) * RPT
UNROLL = 6


@functools.cache
def _dual_spmv_build():
    mesh = plsc.VectorSubcoreMesh(
        core_axis_name="c", subcore_axis_name="s",
        num_cores=NC, num_subcores=NS)
    return pl.kernel(
        _dual_spmv_sc,
        out_type=[jax.ShapeDtypeStruct((N, D), jnp.float32),
                  jax.ShapeDtypeStruct((N, D), jnp.float32)],
        mesh=mesh,
        scratch_types=[
            [pltpu.VMEM((CH,), jnp.int32) for _ in range(3)],
            [pltpu.VMEM((CH,), jnp.int32) for _ in range(3)],
            [pltpu.VMEM((CH, D), jnp.float32) for _ in range(2)],
            pltpu.VMEM_SHARED((NPAD, D), jnp.float32),
            [pltpu.SemaphoreType.DMA for _ in range(3)],
            [pltpu.SemaphoreType.DMA for _ in range(3)],
            [pltpu.SemaphoreType.DMA for _ in range(2)],
        ],
    )


def _dual_spmv_sc(ha_hbm, adja_hbm, hb_hbm, adjb_hbm, zero_hbm,
                  outa_hbm, outb_hbm,
                  sidx, didx, rows, acc_sh, ssem, dsem, gsem):
    cid = lax.axis_index("c")
    sid = lax.axis_index("s")

    def run_branch(h_hbm, adj_hbm, out_hbm):
        ebase = sid * EPT

        def fetch(i, ib):
            off = ebase + i * CH
            pltpu.async_copy(adj_hbm.at[pl.ds(E + off, CH)],
                             sidx[ib], ssem[ib])
            pltpu.async_copy(adj_hbm.at[pl.ds(off, CH)], didx[ib], dsem[ib])

        def fetch_wait(i, ib):
            off = ebase + i * CH
            pltpu.make_async_copy(adj_hbm.at[pl.ds(E + off, CH)],
                                  sidx[ib], ssem[ib]).wait()
            pltpu.make_async_copy(adj_hbm.at[pl.ds(off, CH)],
                                  didx[ib], dsem[ib]).wait()

        def gstart(ib, rb):
            pltpu.async_copy(h_hbm.at[sidx[ib]], rows[rb], gsem[rb])

        def gwait(ib, rb):
            pltpu.make_async_copy(h_hbm.at[sidx[ib]],
                                  rows[rb], gsem[rb]).wait()

        def scat(ib, rb):
            pltpu.sync_copy(rows[rb], acc_sh.at[didx[ib]], add=True)

        fetch(0, 0)
        fetch(1, 1)
        @pl.when(sid < NS - 1)
        def _():
            pltpu.sync_copy(zero_hbm.at[pl.ds(sid * RPT, RPT)],
                            acc_sh.at[pl.ds(sid * RPT, RPT)])

        @pl.when(sid == NS - 1)
        def _():
            pltpu.sync_copy(zero_hbm.at[pl.ds((NS - 1) * RPT, RLAST)],
                            acc_sh.at[pl.ds((NS - 1) * RPT, RLAST)])

        fetch_wait(0, 0)
        gstart(0, 0)
        plsc.subcore_barrier()

        def body(k, _):
            for u in range(UNROLL):
                i = UNROLL * k + u
                fetch(i + 2, (u + 2) % 3)
                fetch_wait(i + 1, (u + 1) % 3)
                gstart((u + 1) % 3, (u + 1) % 2)
                gwait(u % 3, u % 2)
                scat(u % 3, u % 2)
            return ()

        nk = (NCHUNK - UNROLL) // UNROLL
        lax.fori_loop(0, nk, body, ())
        for i in range(NCHUNK - UNROLL, NCHUNK):
            u = i % UNROLL
            if i + 2 < NCHUNK:
                fetch(i + 2, (u + 2) % 3)
            if i + 1 < NCHUNK:
                fetch_wait(i + 1, (u + 1) % 3)
                gstart((u + 1) % 3, (u + 1) % 2)
            gwait(u % 3, u % 2)
            scat(u % 3, u % 2)

        toff = ebase + NCHUNK * CH
        pltpu.async_copy(adj_hbm.at[pl.ds(E + toff, TAIL)],
                         sidx[0].at[pl.ds(0, TAIL)], ssem[0])
        pltpu.async_copy(adj_hbm.at[pl.ds(toff, TAIL)],
                         didx[0].at[pl.ds(0, TAIL)], dsem[0])
        pltpu.make_async_copy(adj_hbm.at[pl.ds(E + toff, TAIL)],
                              sidx[0].at[pl.ds(0, TAIL)], ssem[0]).wait()
        pltpu.make_async_copy(adj_hbm.at[pl.ds(toff, TAIL)],
                              didx[0].at[pl.ds(0, TAIL)], dsem[0]).wait()
        pltpu.async_copy(h_hbm.at[sidx[0].at[pl.ds(0, TAIL)]],
                         rows[0].at[pl.ds(0, TAIL)], gsem[0])
        pltpu.make_async_copy(h_hbm.at[sidx[0].at[pl.ds(0, TAIL)]],
                              rows[0].at[pl.ds(0, TAIL)], gsem[0]).wait()
        pltpu.sync_copy(rows[0].at[pl.ds(0, TAIL)],
                        acc_sh.at[didx[0].at[pl.ds(0, TAIL)]], add=True)

        plsc.subcore_barrier()
        @pl.when(sid < NS - 1)
        def _():
            pltpu.sync_copy(acc_sh.at[pl.ds(sid * RPT, RPT)],
                            out_hbm.at[pl.ds(sid * RPT, RPT)])

        @pl.when(sid == NS - 1)
        def _():
            pltpu.sync_copy(acc_sh.at[pl.ds((NS - 1) * RPT, RLAST)],
                            out_hbm.at[pl.ds((NS - 1) * RPT, RLAST)])

    @pl.when(cid == 0)
    def _():
        run_branch(ha_hbm, adja_hbm, outa_hbm)

    @pl.when(cid == 1)
    def _():
        run_branch(hb_hbm, adjb_hbm, outb_hbm)


def _dual_spmv(ha, adja, hb, adjb, zeros):
    return _dual_spmv_build()(ha, adja, hb, adjb, zeros)


_RBLK = 2000


def _pair_body(xa_ref, wa_ref, ba_ref, xb_ref, wb_ref, bb_ref,
               oa_ref, ob_ref):
    oa_ref[...] = jnp.maximum(
        jnp.dot(xa_ref[...], wa_ref[...],
                preferred_element_type=jnp.float32) + ba_ref[...], 0.0)
    ob_ref[...] = jnp.maximum(
        jnp.dot(xb_ref[...], wb_ref[...],
                preferred_element_type=jnp.float32) + bb_ref[...], 0.0)


def _dense_pair(xa, Wa, ba, xb, Wb, bb):
    xspec = pl.BlockSpec((_RBLK, D), lambda i: (i, 0))
    wspec = pl.BlockSpec((D, D), lambda i: (0, 0))
    bspec = pl.BlockSpec((1, D), lambda i: (0, 0))
    ospec = pl.BlockSpec((_RBLK, D), lambda i: (i, 0))
    oshape = jax.ShapeDtypeStruct((N, D), jnp.float32)
    return pl.pallas_call(
        _pair_body,
        grid=(N // _RBLK,),
        in_specs=[xspec, wspec, bspec, xspec, wspec, bspec],
        out_specs=[ospec, ospec],
        out_shape=[oshape, oshape],
    )(xa, Wa, ba[None, :], xb, Wb, bb[None, :])


def _final_body(xa_ref, h0_ref, h1_ref, qb_ref, wm0_ref, wm1_ref, wm2_ref,
                bm_ref, woa_ref, wob_ref, bo_ref, o_ref):
    xb = jnp.maximum(
        jnp.dot(h0_ref[...], wm0_ref[...], preferred_element_type=jnp.float32)
        + jnp.dot(h1_ref[...], wm1_ref[...],
                  preferred_element_type=jnp.float32)
        + jnp.dot(qb_ref[...], wm2_ref[...],
                  preferred_element_type=jnp.float32)
        + bm_ref[...], 0.0)
    o_ref[...] = (
        jnp.dot(xa_ref[...], woa_ref[...], preferred_element_type=jnp.float32)
        + jnp.dot(xb, wob_ref[...], preferred_element_type=jnp.float32)
        + bo_ref[...])


def _final(xa, h0, h1, qb, Wm0, Wm1, Wm2, bm, Woa, Wob, bo):
    xspec = pl.BlockSpec((_RBLK, D), lambda i: (i, 0))
    wspec = pl.BlockSpec((D, D), lambda i: (0, 0))
    bspec = pl.BlockSpec((1, D), lambda i: (0, 0))
    return pl.pallas_call(
        _final_body,
        grid=(N // _RBLK,),
        in_specs=[xspec, xspec, xspec, xspec, wspec, wspec, wspec, bspec,
                  wspec, wspec, bspec],
        out_specs=pl.BlockSpec((_RBLK, D), lambda i: (i, 0)),
        out_shape=jax.ShapeDtypeStruct((N, D), jnp.float32),
    )(xa, h0, h1, qb, Wm0, Wm1, Wm2, bm[None, :], Woa, Wob, bo[None, :])


def kernel(x, adj_a, adj_b, Wa0, ba0, Wa1, ba1, Wb0, bb0, Wb1, bb1,
           Wm, bm, Wo, bo):
    zeros = jnp.zeros((N, D), jnp.float32)
    adja = adj_a.reshape(2 * E)
    adjb = adj_b.reshape(2 * E)
    Wm0, Wm1, Wm2 = Wm[:D], Wm[D:2 * D], Wm[2 * D:]
    Wo_a, Wo_b = Wo[:D], Wo[D:]

    ha0, hb0 = _dense_pair(x, Wa0, ba0, x, Wb0, bb0)
    pa, pb = _dual_spmv(ha0, adja, hb0, adjb, zeros)
    ha1, hb1 = _dense_pair(pa, Wa1, ba1, pb, Wb1, bb1)
    xa, qb = _dual_spmv(ha1, adja, hb1, adjb, zeros)
    return _final(xa, hb0, hb1, qb, Wm0, Wm1, Wm2, bm, Wo_a, Wo_b, bo)


# speedup vs baseline: 1.4727x; 1.0033x over previous
import functools

import jax
import jax.numpy as jnp
from jax import lax
from jax.experimental import pallas as pl
from jax.experimental.pallas import tpu as pltpu
from jax.experimental.pallas import tpu_sc as plsc

N = 10000
E = 320000
D = 128

NC = 2
NS = 16
EPT = E // NS
CH = 128
NCHUNK = EPT // CH
TAIL = EPT - NCHUNK * CH
NPAD = 10112
RPT = NPAD // NS
RLAST = N - (NS - 1) * RPT
UNROLL = 6


@functools.cache
def _dual_spmv_build():
    mesh = plsc.VectorSubcoreMesh(
        core_axis_name="c", subcore_axis_name="s",
        num_cores=NC, num_subcores=NS)
    return pl.kernel(
        _dual_spmv_sc,
        out_type=[jax.ShapeDtypeStruct((N, D), jnp.float32),
                  jax.ShapeDtypeStruct((N, D), jnp.float32)],
        mesh=mesh,
        scratch_types=[
            [pltpu.VMEM((CH,), jnp.int32) for _ in range(3)],
            [pltpu.VMEM((CH,), jnp.int32) for _ in range(3)],
            [pltpu.VMEM((CH, D), jnp.float32) for _ in range(2)],
            pltpu.VMEM_SHARED((NPAD, D), jnp.float32),
            [pltpu.SemaphoreType.DMA for _ in range(3)],
            [pltpu.SemaphoreType.DMA for _ in range(3)],
            [pltpu.SemaphoreType.DMA for _ in range(2)],
        ],
    )


def _dual_spmv_sc(ha_hbm, adja_hbm, hb_hbm, adjb_hbm, zero_hbm,
                  outa_hbm, outb_hbm,
                  sidx, didx, rows, acc_sh, ssem, dsem, gsem):
    cid = lax.axis_index("c")
    sid = lax.axis_index("s")

    def run_branch(h_hbm, adj_hbm, out_hbm):
        ebase = sid * EPT

        def fetch(i, ib):
            off = ebase + i * CH
            pltpu.async_copy(adj_hbm.at[pl.ds(E + off, CH)],
                             sidx[ib], ssem[ib])
            pltpu.async_copy(adj_hbm.at[pl.ds(off, CH)], didx[ib], dsem[ib])

        def fetch_wait(i, ib):
            off = ebase + i * CH
            pltpu.make_async_copy(adj_hbm.at[pl.ds(E + off, CH)],
                                  sidx[ib], ssem[ib]).wait()
            pltpu.make_async_copy(adj_hbm.at[pl.ds(off, CH)],
                                  didx[ib], dsem[ib]).wait()

        def gstart(ib, rb):
            pltpu.async_copy(h_hbm.at[sidx[ib]], rows[rb], gsem[rb])

        def gwait(ib, rb):
            pltpu.make_async_copy(h_hbm.at[sidx[ib]],
                                  rows[rb], gsem[rb]).wait()

        def scat(ib, rb):
            pltpu.sync_copy(rows[rb], acc_sh.at[didx[ib]], add=True)

        fetch(0, 0)
        fetch(1, 1)
        @pl.when(sid < NS - 1)
        def _():
            pltpu.sync_copy(zero_hbm.at[pl.ds(0, RPT)],
                            acc_sh.at[pl.ds(sid * RPT, RPT)])

        @pl.when(sid == NS - 1)
        def _():
            pltpu.sync_copy(zero_hbm.at[pl.ds(0, RLAST)],
                            acc_sh.at[pl.ds((NS - 1) * RPT, RLAST)])

        fetch_wait(0, 0)
        gstart(0, 0)
        plsc.subcore_barrier()

        def body(k, _):
            for u in range(UNROLL):
                i = UNROLL * k + u
                fetch(i + 2, (u + 2) % 3)
                fetch_wait(i + 1, (u + 1) % 3)
                gstart((u + 1) % 3, (u + 1) % 2)
                gwait(u % 3, u % 2)
                scat(u % 3, u % 2)
            return ()

        nk = (NCHUNK - UNROLL) // UNROLL
        lax.fori_loop(0, nk, body, ())
        for i in range(NCHUNK - UNROLL, NCHUNK):
            u = i % UNROLL
            if i + 2 < NCHUNK:
                fetch(i + 2, (u + 2) % 3)
            if i + 1 < NCHUNK:
                fetch_wait(i + 1, (u + 1) % 3)
                gstart((u + 1) % 3, (u + 1) % 2)
            gwait(u % 3, u % 2)
            scat(u % 3, u % 2)

        toff = ebase + NCHUNK * CH
        pltpu.async_copy(adj_hbm.at[pl.ds(E + toff, TAIL)],
                         sidx[0].at[pl.ds(0, TAIL)], ssem[0])
        pltpu.async_copy(adj_hbm.at[pl.ds(toff, TAIL)],
                         didx[0].at[pl.ds(0, TAIL)], dsem[0])
        pltpu.make_async_copy(adj_hbm.at[pl.ds(E + toff, TAIL)],
                              sidx[0].at[pl.ds(0, TAIL)], ssem[0]).wait()
        pltpu.make_async_copy(adj_hbm.at[pl.ds(toff, TAIL)],
                              didx[0].at[pl.ds(0, TAIL)], dsem[0]).wait()
        pltpu.async_copy(h_hbm.at[sidx[0].at[pl.ds(0, TAIL)]],
                         rows[0].at[pl.ds(0, TAIL)], gsem[0])
        pltpu.make_async_copy(h_hbm.at[sidx[0].at[pl.ds(0, TAIL)]],
                              rows[0].at[pl.ds(0, TAIL)], gsem[0]).wait()
        pltpu.sync_copy(rows[0].at[pl.ds(0, TAIL)],
                        acc_sh.at[didx[0].at[pl.ds(0, TAIL)]], add=True)

        plsc.subcore_barrier()
        @pl.when(sid < NS - 1)
        def _():
            pltpu.sync_copy(acc_sh.at[pl.ds(sid * RPT, RPT)],
                            out_hbm.at[pl.ds(sid * RPT, RPT)])

        @pl.when(sid == NS - 1)
        def _():
            pltpu.sync_copy(acc_sh.at[pl.ds((NS - 1) * RPT, RLAST)],
                            out_hbm.at[pl.ds((NS - 1) * RPT, RLAST)])

    @pl.when(cid == 0)
    def _():
        run_branch(ha_hbm, adja_hbm, outa_hbm)

    @pl.when(cid == 1)
    def _():
        run_branch(hb_hbm, adjb_hbm, outb_hbm)


def _dual_spmv(ha, adja, hb, adjb, zeros):
    return _dual_spmv_build()(ha, adja, hb, adjb, zeros)


_RBLK = 2000


def _pair_body(xa_ref, wa_ref, ba_ref, xb_ref, wb_ref, bb_ref,
               oa_ref, ob_ref):
    oa_ref[...] = jnp.maximum(
        jnp.dot(xa_ref[...], wa_ref[...],
                preferred_element_type=jnp.float32) + ba_ref[...], 0.0)
    ob_ref[...] = jnp.maximum(
        jnp.dot(xb_ref[...], wb_ref[...],
                preferred_element_type=jnp.float32) + bb_ref[...], 0.0)


def _pair1_body(x_ref, wa_ref, ba_ref, wb_ref, bb_ref, oa_ref, ob_ref):
    x = x_ref[...]
    oa_ref[...] = jnp.maximum(
        jnp.dot(x, wa_ref[...],
                preferred_element_type=jnp.float32) + ba_ref[...], 0.0)
    ob_ref[...] = jnp.maximum(
        jnp.dot(x, wb_ref[...],
                preferred_element_type=jnp.float32) + bb_ref[...], 0.0)


_xspec = pl.BlockSpec((_RBLK, D), lambda i: (i, 0))
_wspec = pl.BlockSpec((D, D), lambda i: (0, 0))
_bspec = pl.BlockSpec((1, D), lambda i: (0, 0))
_oshape = jax.ShapeDtypeStruct((N, D), jnp.float32)


def _dense_pair(xa, Wa, ba, xb, Wb, bb):
    return pl.pallas_call(
        _pair_body,
        grid=(N // _RBLK,),
        in_specs=[_xspec, _wspec, _bspec, _xspec, _wspec, _bspec],
        out_specs=[_xspec, _xspec],
        out_shape=[_oshape, _oshape],
    )(xa, Wa, ba[None, :], xb, Wb, bb[None, :])


def _dense_pair1(x, Wa, ba, Wb, bb):
    return pl.pallas_call(
        _pair1_body,
        grid=(N // _RBLK,),
        in_specs=[_xspec, _wspec, _bspec, _wspec, _bspec],
        out_specs=[_xspec, _xspec],
        out_shape=[_oshape, _oshape],
    )(x, Wa, ba[None, :], Wb, bb[None, :])


def _final_body(xa_ref, h0_ref, h1_ref, qb_ref, wm0_ref, wm1_ref, wm2_ref,
                bm_ref, woa_ref, wob_ref, bo_ref, o_ref):
    xb = jnp.maximum(
        jnp.dot(h0_ref[...], wm0_ref[...], preferred_element_type=jnp.float32)
        + jnp.dot(h1_ref[...], wm1_ref[...],
                  preferred_element_type=jnp.float32)
        + jnp.dot(qb_ref[...], wm2_ref[...],
                  preferred_element_type=jnp.float32)
        + bm_ref[...], 0.0)
    o_ref[...] = (
        jnp.dot(xa_ref[...], woa_ref[...], preferred_element_type=jnp.float32)
        + jnp.dot(xb, wob_ref[...], preferred_element_type=jnp.float32)
        + bo_ref[...])


def _final(xa, h0, h1, qb, Wm0, Wm1, Wm2, bm, Woa, Wob, bo):
    xspec = pl.BlockSpec((_RBLK, D), lambda i: (i, 0))
    wspec = pl.BlockSpec((D, D), lambda i: (0, 0))
    bspec = pl.BlockSpec((1, D), lambda i: (0, 0))
    return pl.pallas_call(
        _final_body,
        grid=(N // _RBLK,),
        in_specs=[xspec, xspec, xspec, xspec, wspec, wspec, wspec, bspec,
                  wspec, wspec, bspec],
        out_specs=pl.BlockSpec((_RBLK, D), lambda i: (i, 0)),
        out_shape=jax.ShapeDtypeStruct((N, D), jnp.float32),
    )(xa, h0, h1, qb, Wm0, Wm1, Wm2, bm[None, :], Woa, Wob, bo[None, :])


def kernel(x, adj_a, adj_b, Wa0, ba0, Wa1, ba1, Wb0, bb0, Wb1, bb1,
           Wm, bm, Wo, bo):
    zeros = jnp.zeros((RPT, D), jnp.float32)
    adja = adj_a.reshape(2 * E)
    adjb = adj_b.reshape(2 * E)
    Wm0, Wm1, Wm2 = Wm[:D], Wm[D:2 * D], Wm[2 * D:]
    Wo_a, Wo_b = Wo[:D], Wo[D:]

    ha0, hb0 = _dense_pair1(x, Wa0, ba0, Wb0, bb0)
    pa, pb = _dual_spmv(ha0, adja, hb0, adjb, zeros)
    ha1, hb1 = _dense_pair(pa, Wa1, ba1, pb, Wb1, bb1)
    xa, qb = _dual_spmv(ha1, adja, hb1, adjb, zeros)
    return _final(xa, hb0, hb1, qb, Wm0, Wm1, Wm2, bm, Wo_a, Wo_b, bo)
